# Initial kernel scaffold; baseline (speedup 1.0000x reference)
import jax
import jax.numpy as jnp
from jax.experimental import pallas as pl


def kernel(x_src, x_dst, edge_index, edge_attr, params):
    raise NotImplementedError("write your pallas kernel here")


# speedup vs baseline: 2.8973x; 2.8973x over previous
import functools

import jax
import jax.numpy as jnp
from jax import lax
from jax.experimental import pallas as pl
from jax.experimental.pallas import tpu as pltpu
from jax.experimental.pallas import tpu_sc as plsc

N = 10000
E = 160000
H = 128
EDIM = 16

NC = 2
NS = 16
NW = NC * NS
EPW = E // NW
CH = 128
NFULL = EPW // CH
TAIL = EPW - NFULL * CH
RPT = 624
REXTRA = N - RPT * NS

BR = 2000


def _silu(x):
    return x * (1.0 / (1.0 + jnp.exp(-x)))


def _ln(h, g, beta):
    mu = jnp.mean(h, axis=-1, keepdims=True)
    var = jnp.mean((h - mu) ** 2, axis=-1, keepdims=True)
    return (h - mu) * jax.lax.rsqrt(var + 1e-5) * g + beta


def _dot(a, b):
    return jnp.dot(a, b, preferred_element_type=jnp.float32)


def _edge_enc_body(x_ref, w0, b0, w1, b1, w2, b2, g, beta, out_ref):
    h = _silu(_dot(x_ref[...], w0[...]) + b0[...])
    h = _silu(_dot(h, w1[...]) + b1[...])
    h = _dot(h, w2[...]) + b2[...]
    out_ref[...] = _ln(h, g[...], beta[...])


def _edge_enc(edge_attr, p):
    w0, b0, w1, b1, w2, b2, g, beta = p
    nb = E // BR
    row = pl.BlockSpec((BR, None), lambda i: (i, 0))
    full = lambda a: pl.BlockSpec(a.shape, lambda i: tuple(0 for _ in a.shape))
    args = (w0, b0.reshape(1, H), w1, b1.reshape(1, H), w2, b2.reshape(1, H),
            g.reshape(1, H), beta.reshape(1, H))
    return pl.pallas_call(
        _edge_enc_body,
        grid=(nb,),
        in_specs=[pl.BlockSpec((BR, EDIM), lambda i: (i, 0))] + [full(a) for a in args],
        out_specs=pl.BlockSpec((BR, H), lambda i: (i, 0)),
        out_shape=jax.ShapeDtypeStruct((E, H), jnp.float32),
    )(edge_attr, *args)


def _project_body(x_ref, w_ref, out_ref):
    out_ref[...] = _dot(x_ref[...], w_ref[...])


def _project(x, w):
    return pl.pallas_call(
        _project_body,
        in_specs=[pl.BlockSpec((N, H), lambda: (0, 0)),
                  pl.BlockSpec((H, H), lambda: (0, 0))],
        out_specs=pl.BlockSpec((N, H), lambda: (0, 0)),
        out_shape=jax.ShapeDtypeStruct((N, H), jnp.float32),
    )(x, w)


def _edge_mlp_body(gi_ref, xj_ref, ea_ref, w0j, w0e, b0, w1, b1, w2, b2, g,
                   beta, out_ref):
    ea = ea_ref[...]
    h = _silu(gi_ref[...] + _dot(xj_ref[...], w0j[...]) + _dot(ea, w0e[...])
              + b0[...])
    h = _silu(_dot(h, w1[...]) + b1[...])
    h = _dot(h, w2[...]) + b2[...]
    out_ref[...] = _ln(h, g[...], beta[...]) + ea


def _edge_mlp(gath_i, xj, ea, p):
    w0, b0, w1, b1, w2, b2, g, beta = p
    w0j = w0[H:2 * H]
    w0e = w0[2 * H:]
    nb = E // BR
    full = lambda a: pl.BlockSpec(a.shape, lambda i: tuple(0 for _ in a.shape))
    args = (w0j, w0e, b0.reshape(1, H), w1, b1.reshape(1, H), w2,
            b2.reshape(1, H), g.reshape(1, H), beta.reshape(1, H))
    row = pl.BlockSpec((BR, H), lambda i: (i, 0))
    return pl.pallas_call(
        _edge_mlp_body,
        grid=(nb,),
        in_specs=[row, row, row] + [full(a) for a in args],
        out_specs=row,
        out_shape=jax.ShapeDtypeStruct((E, H), jnp.float32),
    )(gath_i, xj, ea, *args)


def _node_mlp_body(xd_ref, a0_ref, a1_ref, w0a, w0b, b0, w1, b1, w2, b2, g,
                   beta, out_ref):
    xd = xd_ref[...]
    agg = a0_ref[...] + a1_ref[...]
    h = _silu(_dot(xd, w0a[...]) + _dot(agg, w0b[...]) + b0[...])
    h = _silu(_dot(h, w1[...]) + b1[...])
    h = _dot(h, w2[...]) + b2[...]
    out_ref[...] = _ln(h, g[...], beta[...]) + xd


def _node_mlp(xd, agg2, p):
    w0, b0, w1, b1, w2, b2, g, beta = p
    w0a = w0[:H]
    w0b = w0[H:]
    nbr = 2000
    nb = N // nbr
    full = lambda a: pl.BlockSpec(a.shape, lambda i: tuple(0 for _ in a.shape))
    args = (w0a, w0b, b0.reshape(1, H), w1, b1.reshape(1, H), w2,
            b2.reshape(1, H), g.reshape(1, H), beta.reshape(1, H))
    row = pl.BlockSpec((nbr, H), lambda i: (i, 0))
    return pl.pallas_call(
        _node_mlp_body,
        grid=(nb,),
        in_specs=[row, row, row] + [full(a) for a in args],
        out_specs=row,
        out_shape=jax.ShapeDtypeStruct((N, H), jnp.float32),
    )(xd, agg2[0], agg2[1], *args)


def _sc_mesh():
    return plsc.VectorSubcoreMesh(core_axis_name="c", subcore_axis_name="s",
                                  num_cores=NC, num_subcores=NS)


def _gather_rows(table, idx):
    @functools.partial(
        pl.kernel,
        out_type=jax.ShapeDtypeStruct((E, H), jnp.float32),
        mesh=_sc_mesh(),
        scratch_types=[
            pltpu.VMEM((CH,), jnp.int32),
            pltpu.VMEM((TAIL,), jnp.int32),
            pltpu.VMEM((CH, H), jnp.float32),
            pltpu.VMEM((TAIL, H), jnp.float32),
            pltpu.SemaphoreType.DMA,
        ],
    )
    def k(table_hbm, idx_hbm, out_hbm, idx_v, idxt_v, rows_v, rowst_v, sem):
        wid = lax.axis_index("s") * NC + lax.axis_index("c")
        base = wid * EPW
        for j in range(NFULL):
            off = base + j * CH
            pltpu.sync_copy(idx_hbm.at[pl.ds(off, CH)], idx_v)
            pltpu.async_copy(table_hbm.at[idx_v], rows_v, sem).wait()
            pltpu.sync_copy(rows_v, out_hbm.at[pl.ds(off, CH)])
        off = base + NFULL * CH
        pltpu.sync_copy(idx_hbm.at[pl.ds(off, TAIL)], idxt_v)
        pltpu.async_copy(table_hbm.at[idxt_v], rowst_v, sem).wait()
        pltpu.sync_copy(rowst_v, out_hbm.at[pl.ds(off, TAIL)])

    return k(table, idx)


def _scatter_add(rows, idx, zeros_tbl):
    @functools.partial(
        pl.kernel,
        out_type=jax.ShapeDtypeStruct((NC, N, H), jnp.float32),
        mesh=_sc_mesh(),
        scratch_types=[
            pltpu.VMEM_SHARED((N, H), jnp.float32),
            pltpu.VMEM((CH,), jnp.int32),
            pltpu.VMEM((TAIL,), jnp.int32),
            pltpu.VMEM((CH, H), jnp.float32),
            pltpu.VMEM((TAIL, H), jnp.float32),
        ],
    )
    def k(rows_hbm, idx_hbm, zeros_hbm, out_hbm, acc, idx_v, idxt_v, rows_v,
          rowst_v):
        cid = lax.axis_index("c")
        sid = lax.axis_index("s")
        wid = sid * NC + cid
        pltpu.sync_copy(zeros_hbm.at[pl.ds(sid * RPT, RPT)],
                        acc.at[pl.ds(sid * RPT, RPT)])

        @pl.when(sid == NS - 1)
        def _():
            pltpu.sync_copy(zeros_hbm.at[pl.ds(RPT * NS, REXTRA)],
                            acc.at[pl.ds(RPT * NS, REXTRA)])

        plsc.subcore_barrier()
        base = wid * EPW
        for j in range(NFULL):
            off = base + j * CH
            pltpu.sync_copy(idx_hbm.at[pl.ds(off, CH)], idx_v)
            pltpu.sync_copy(rows_hbm.at[pl.ds(off, CH)], rows_v)
            pltpu.sync_copy(rows_v, acc.at[idx_v], add=True)
        off = base + NFULL * CH
        pltpu.sync_copy(idx_hbm.at[pl.ds(off, TAIL)], idxt_v)
        pltpu.sync_copy(rows_hbm.at[pl.ds(off, TAIL)], rowst_v)
        pltpu.sync_copy(rowst_v, acc.at[idxt_v], add=True)
        plsc.subcore_barrier()
        pltpu.sync_copy(acc.at[pl.ds(sid * RPT, RPT)],
                        out_hbm.at[cid, pl.ds(sid * RPT, RPT)])

        @pl.when(sid == NS - 1)
        def _():
            pltpu.sync_copy(acc.at[pl.ds(RPT * NS, REXTRA)],
                            out_hbm.at[cid, pl.ds(RPT * NS, REXTRA)])

    return k(rows, idx, zeros_tbl)


def kernel(x_src, x_dst, edge_index, edge_attr, params):
    src = edge_index[0]
    dst = edge_index[1]
    zeros_tbl = jnp.zeros((N, H), jnp.float32)

    ea = _edge_enc(edge_attr, params["edge_enc"])
    xj = _gather_rows(x_src, src)

    xd = x_dst
    for blk in params["blocks"]:
        ep = blk["edge_mlp"]
        w0i = ep[0][:H]
        pre_i = _project(xd, w0i)
        gath_i = _gather_rows(pre_i, dst)
        e_new = _edge_mlp(gath_i, xj, ea, ep)
        agg2 = _scatter_add(e_new, dst, zeros_tbl)
        xd = _node_mlp(xd, agg2, blk["node_mlp"])
        ea = e_new
    return xd
